# baseline (speedup 1.0000x reference)
import functools

import jax
import jax.numpy as jnp
from jax import lax
from jax.experimental import pallas as pl
from jax.experimental.pallas import tpu as pltpu

_DECAY_MIN = 0.9
_DECAY_MAX = 0.999

_C = 256
_K = 8


def _phase1(scal_ref, x_ref, w_ref, intra_ref, qd_ref, ktv_ref,
            mask_scr, pow_scr, *, M):
    c = pl.program_id(1)
    bg = scal_ref[0]
    ld = scal_ref[1]

    @pl.when(jnp.logical_and(pl.program_id(0) == 0, c == 0))
    def _init():
        ti = lax.broadcasted_iota(jnp.int32, (_C, _C), 0).astype(jnp.float32)
        si = lax.broadcasted_iota(jnp.int32, (_C, _C), 1).astype(jnp.float32)
        mask_scr[...] = jnp.where(si <= ti, jnp.exp((ti - si) * ld), 0.0)
        tc = lax.broadcasted_iota(jnp.int32, (_C, 2 * M), 0).astype(jnp.float32)
        li = lax.broadcasted_iota(jnp.int32, (_C, 2 * M), 1)
        pow_scr[...] = jnp.where(li < M,
                                 jnp.exp((tc + 1.0) * ld),
                                 jnp.exp((_C - 1.0 - tc) * ld))

    mask = mask_scr[...]
    dpow = pow_scr[:, 0:M]
    rpow = pow_scr[:, M:2 * M]

    for j in range(_K):
        xb = x_ref[0, j * _C:(j + 1) * _C, :]
        vkqg = jnp.dot(xb, w_ref[...], preferred_element_type=jnp.float32)
        v = vkqg[:, 0:M]
        k = vkqg[:, M:2 * M]
        q = vkqg[:, 2 * M:3 * M]
        g = jax.nn.sigmoid(vkqg[:, 3 * M:4 * M] + bg)
        g2 = g * g
        gk2 = k * g2

        a = lax.dot_general(q, gk2, (((1,), (1,)), ((), ())),
                            preferred_element_type=jnp.float32)
        am = a * mask
        intra_ref[0, j * _C:(j + 1) * _C, :] = jnp.dot(
            am, v, preferred_element_type=jnp.float32)

        qd_ref[0, j * _C:(j + 1) * _C, :] = (q * dpow).astype(jnp.bfloat16)

        ktv_ref[0, j] = lax.dot_general(
            gk2 * rpow, v, (((0,), (0,)), ((), ())),
            preferred_element_type=jnp.float32)


def _phase2(scal_ref, ktv_ref, sall_ref, sfin_ref, *, nc, M):
    decay_c = scal_ref[2]
    s = jnp.zeros((M, M), jnp.float32)
    for c in range(nc):
        sall_ref[0, c] = s
        s = decay_c * s + ktv_ref[0, c]
    sfin_ref[0] = s


def _phase3(intra_ref, qd_ref, sall_ref, wo_ref, bo_ref, y_ref, *, M):
    for j in range(_K):
        inter = jnp.dot(qd_ref[0, j * _C:(j + 1) * _C, :], sall_ref[0, j],
                        preferred_element_type=jnp.float32)
        out = intra_ref[0, j * _C:(j + 1) * _C, :] + inter
        y_ref[0, j * _C:(j + 1) * _C, :] = (
            jnp.dot(out, wo_ref[...], preferred_element_type=jnp.float32)
            + bo_ref[...])


def kernel(x, Wq, Wk, Wv, Wo, bo, Wg, bg, decay_param):
    B, L, D = x.shape
    M = Wq.shape[1]
    nsteps = L // (_C * _K)
    nc = L // _C

    decay = _DECAY_MIN + jax.nn.sigmoid(decay_param[0]) * (_DECAY_MAX - _DECAY_MIN)
    ld = jnp.log(decay)
    scal = jnp.stack([bg[0], ld, decay ** _C]).astype(jnp.float32)

    w_all = jnp.concatenate([Wv, Wk, Wq, jnp.tile(Wg, (1, M))], axis=1)
    bo2 = bo.reshape(1, D)

    p1 = functools.partial(_phase1, M=M)
    intra, qd, ktv = pl.pallas_call(
        p1,
        grid=(B, nsteps),
        in_specs=[
            pl.BlockSpec(memory_space=pltpu.SMEM),
            pl.BlockSpec((1, _C * _K, D), lambda b, c: (b, c, 0)),
            pl.BlockSpec((D, 4 * M), lambda b, c: (0, 0)),
        ],
        out_specs=[
            pl.BlockSpec((1, _C * _K, M), lambda b, c: (b, c, 0)),
            pl.BlockSpec((1, _C * _K, M), lambda b, c: (b, c, 0)),
            pl.BlockSpec((1, _K, M, M), lambda b, c: (b, c, 0, 0)),
        ],
        out_shape=[
            jax.ShapeDtypeStruct((B, L, M), jnp.float32),
            jax.ShapeDtypeStruct((B, L, M), jnp.bfloat16),
            jax.ShapeDtypeStruct((B, nc, M, M), jnp.float32),
        ],
        scratch_shapes=[
            pltpu.VMEM((_C, _C), jnp.float32),
            pltpu.VMEM((_C, 2 * M), jnp.float32),
        ],
        compiler_params=pltpu.CompilerParams(
            dimension_semantics=("arbitrary", "arbitrary"),
            vmem_limit_bytes=56 * 1024 * 1024,
        ),
    )(scal, x, w_all)

    p2 = functools.partial(_phase2, nc=nc, M=M)
    s_all, s_final = pl.pallas_call(
        p2,
        grid=(B,),
        in_specs=[
            pl.BlockSpec(memory_space=pltpu.SMEM),
            pl.BlockSpec((1, nc, M, M), lambda b: (b, 0, 0, 0)),
        ],
        out_specs=[
            pl.BlockSpec((1, nc, M, M), lambda b: (b, 0, 0, 0)),
            pl.BlockSpec((1, M, M), lambda b: (b, 0, 0)),
        ],
        out_shape=[
            jax.ShapeDtypeStruct((B, nc, M, M), jnp.float32),
            jax.ShapeDtypeStruct((B, M, M), jnp.float32),
        ],
    )(scal, ktv)

    p3 = functools.partial(_phase3, M=M)
    y = pl.pallas_call(
        p3,
        grid=(B, nsteps),
        in_specs=[
            pl.BlockSpec((1, _C * _K, M), lambda b, c: (b, c, 0)),
            pl.BlockSpec((1, _C * _K, M), lambda b, c: (b, c, 0)),
            pl.BlockSpec((1, _K, M, M), lambda b, c: (b, c, 0, 0)),
            pl.BlockSpec((M, D), lambda b, c: (0, 0)),
            pl.BlockSpec((1, D), lambda b, c: (0, 0)),
        ],
        out_specs=pl.BlockSpec((1, _C * _K, D), lambda b, c: (b, c, 0)),
        out_shape=jax.ShapeDtypeStruct((B, L, D), jnp.float32),
        compiler_params=pltpu.CompilerParams(
            dimension_semantics=("arbitrary", "arbitrary"),
            vmem_limit_bytes=56 * 1024 * 1024,
        ),
    )(intra, qd, s_all, Wo, bo2)
    return y, s_final


# speedup vs baseline: 1.4047x; 1.4047x over previous
import functools

import jax
import jax.numpy as jnp
from jax import lax
from jax.experimental import pallas as pl
from jax.experimental.pallas import tpu as pltpu

_DECAY_MIN = 0.9
_DECAY_MAX = 0.999

_C = 512
_K = 4


def _body(scal_ref, x_ref, w_ref, wo_ref, bo_ref, y_ref, sfin_ref,
          mask_scr, pow_scr, *, M):
    c = pl.program_id(1)
    bg = scal_ref[0]
    ld = scal_ref[1]
    decay_c = scal_ref[2]

    @pl.when(jnp.logical_and(pl.program_id(0) == 0, c == 0))
    def _init():
        ti = lax.broadcasted_iota(jnp.int32, (_C, _C), 0).astype(jnp.float32)
        si = lax.broadcasted_iota(jnp.int32, (_C, _C), 1).astype(jnp.float32)
        mask_scr[...] = jnp.where(si <= ti, jnp.exp((ti - si) * ld), 0.0)
        tc = lax.broadcasted_iota(jnp.int32, (_C, 2 * M), 0).astype(jnp.float32)
        li = lax.broadcasted_iota(jnp.int32, (_C, 2 * M), 1)
        pow_scr[...] = jnp.where(li < M,
                                 jnp.exp((tc + 1.0) * ld),
                                 jnp.exp((_C - 1.0 - tc) * ld))

    @pl.when(c == 0)
    def _zero_state():
        sfin_ref[...] = jnp.zeros_like(sfin_ref)

    mask = mask_scr[...]
    dpow = pow_scr[:, 0:M]
    rpow = pow_scr[:, M:2 * M]
    s = sfin_ref[0]

    for j in range(_K):
        xb = x_ref[0, j * _C:(j + 1) * _C, :]
        vkqg = jnp.dot(xb, w_ref[...], preferred_element_type=jnp.float32)
        v = vkqg[:, 0:M]
        k = vkqg[:, M:2 * M]
        q = vkqg[:, 2 * M:3 * M]
        g = jax.nn.sigmoid(vkqg[:, 3 * M:4 * M] + bg)
        g2 = g * g
        gk2 = k * g2

        a = lax.dot_general(q, gk2, (((1,), (1,)), ((), ())),
                            preferred_element_type=jnp.float32)
        am = a * mask
        intra = jnp.dot(am, v, preferred_element_type=jnp.float32)

        inter = jnp.dot(q * dpow, s, preferred_element_type=jnp.float32)

        out = intra + inter
        y_ref[0, j * _C:(j + 1) * _C, :] = (
            jnp.dot(out, wo_ref[...], preferred_element_type=jnp.float32)
            + bo_ref[...])

        ktv = lax.dot_general(gk2 * rpow, v, (((0,), (0,)), ((), ())),
                              preferred_element_type=jnp.float32)
        s = decay_c * s + ktv

    sfin_ref[0] = s


def kernel(x, Wq, Wk, Wv, Wo, bo, Wg, bg, decay_param):
    B, L, D = x.shape
    M = Wq.shape[1]
    nsteps = L // (_C * _K)

    decay = _DECAY_MIN + jax.nn.sigmoid(decay_param[0]) * (_DECAY_MAX - _DECAY_MIN)
    ld = jnp.log(decay)
    scal = jnp.stack([bg[0], ld, decay ** _C]).astype(jnp.float32)

    w_all = jnp.concatenate([Wv, Wk, Wq, jnp.tile(Wg, (1, M))], axis=1)
    bo2 = bo.reshape(1, D)

    body = functools.partial(_body, M=M)
    y, s_final = pl.pallas_call(
        body,
        grid=(B, nsteps),
        in_specs=[
            pl.BlockSpec(memory_space=pltpu.SMEM),
            pl.BlockSpec((1, _C * _K, D), lambda b, c: (b, c, 0)),
            pl.BlockSpec((D, 4 * M), lambda b, c: (0, 0)),
            pl.BlockSpec((M, D), lambda b, c: (0, 0)),
            pl.BlockSpec((1, D), lambda b, c: (0, 0)),
        ],
        out_specs=[
            pl.BlockSpec((1, _C * _K, D), lambda b, c: (b, c, 0)),
            pl.BlockSpec((1, M, M), lambda b, c: (b, 0, 0)),
        ],
        out_shape=[
            jax.ShapeDtypeStruct((B, L, D), jnp.float32),
            jax.ShapeDtypeStruct((B, M, M), jnp.float32),
        ],
        scratch_shapes=[
            pltpu.VMEM((_C, _C), jnp.float32),
            pltpu.VMEM((_C, 2 * M), jnp.float32),
        ],
        compiler_params=pltpu.CompilerParams(
            dimension_semantics=("arbitrary", "arbitrary"),
            vmem_limit_bytes=56 * 1024 * 1024,
        ),
    )(scal, x, w_all, Wo, bo2)
    return y, s_final
